# baseline (speedup 1.0000x reference)
import functools

import jax
import jax.numpy as jnp
from jax import lax
from jax.experimental import pallas as pl
from jax.experimental.pallas import tpu as pltpu
from jax.experimental.pallas import tpu_sc as plsc

_SUB = 8
_N_NEG = 512
_B_PNP = 2.0
_ALPHA = 1.0
_ANNEAL = 0.01
_MAGIC = float(1.5 * (2.0 ** 23))

_L = 16


def _sc_detile(feat2, B, C, H, W):
    mesh = plsc.VectorSubcoreMesh(core_axis_name="c", subcore_axis_name="s")
    NC = mesh.num_cores
    NW = NC * mesh.num_subcores
    P = B * C
    PPW = P // NW

    @functools.partial(
        pl.kernel,
        out_type=[jax.ShapeDtypeStruct((P, H, W), jnp.float32)],
        mesh=mesh,
        scratch_types=[pltpu.SemaphoreType.DMA],
    )
    def k(f2_hbm, scr_hbm, sem):
        wid = lax.axis_index("s") * NC + lax.axis_index("c")
        p0 = wid * PPW
        cps = []
        for pi in range(PPW):
            p = p0 + pi
            cps.append(pltpu.async_copy(
                f2_hbm.at[p // C, p % C], scr_hbm.at[p], sem))
        for cp in cps:
            cp.wait()

    return k(feat2)[0]


def _sc_gather(f2flat, targ, nidx, B, C, H, W, Q):
    HW = H * W
    N = _N_NEG
    mesh = plsc.VectorSubcoreMesh(core_axis_name="c", subcore_axis_name="s")
    NC = mesh.num_cores
    NW = NC * mesh.num_subcores
    WPB = NW // B
    CPW = C // WPB
    PCH = 112
    NCH = 128

    @functools.partial(
        pl.kernel,
        out_type=[
            jax.ShapeDtypeStruct((B, C, Q), jnp.float32),
            jax.ShapeDtypeStruct((B, C, N), jnp.float32),
            jax.ShapeDtypeStruct((B, Q), jnp.float32),
        ],
        mesh=mesh,
        scratch_types=[
            pltpu.VMEM((Q,), jnp.float32),
            pltpu.VMEM((Q,), jnp.float32),
            pltpu.VMEM((Q,), jnp.float32),
            pltpu.VMEM((Q,), jnp.int32),
            pltpu.VMEM((N,), jnp.int32),
            pltpu.VMEM((Q,), jnp.float32),
            pltpu.VMEM((N,), jnp.float32),
            pltpu.SemaphoreType.DMA,
        ],
    )
    def k(f2_hbm, targ_hbm, nidx_hbm, pf_hbm, nf_hbm, msk_hbm,
          txv, tyv, mskv, idxv, nidxv, gv, ngv, sem):
        wid = lax.axis_index("s") * NC + lax.axis_index("c")
        b = wid // WPB
        cblk = (wid % WPB) * CPW
        base_off = (b * C + cblk) * HW

        pltpu.sync_copy(targ_hbm.at[b, 0], txv)
        pltpu.sync_copy(targ_hbm.at[b, 1], tyv)
        pltpu.sync_copy(nidx_hbm.at[b], nidxv)

        def idx_body(i, _):
            tx = txv[pl.ds(i * _L, _L)]
            ty = tyv[pl.ds(i * _L, _L)]
            rx = (tx + _MAGIC) - _MAGIC
            ry = (ty + _MAGIC) - _MAGIC
            ok = ((rx >= 0.0) & (rx <= W - 1.0)
                  & (ry >= 0.0) & (ry <= H - 1.0))
            cx = jnp.clip(rx, 0.0, W - 1.0).astype(jnp.int32)
            cy = jnp.clip(ry, 0.0, H - 1.0).astype(jnp.int32)
            mskv[pl.ds(i * _L, _L)] = jnp.where(ok, 1.0, 0.0)
            idxv[pl.ds(i * _L, _L)] = cy * W + cx + base_off
            return 0

        lax.fori_loop(0, Q // _L, idx_body, 0, unroll=True)

        def nidx_body(i, _):
            nidxv[pl.ds(i * _L, _L)] = nidxv[pl.ds(i * _L, _L)] + base_off
            return 0

        lax.fori_loop(0, N // _L, nidx_body, 0, unroll=True)

        @pl.when(cblk == 0)
        def _():
            pltpu.sync_copy(mskv, msk_hbm.at[b])

        def c_body(ci, _):
            c = cblk + ci
            cps = []
            for j in range(Q // PCH):
                cps.append(pltpu.async_copy(
                    f2_hbm.at[idxv.at[pl.ds(j * PCH, PCH)]],
                    gv.at[pl.ds(j * PCH, PCH)], sem))
            for j in range(N // NCH):
                cps.append(pltpu.async_copy(
                    f2_hbm.at[nidxv.at[pl.ds(j * NCH, NCH)]],
                    ngv.at[pl.ds(j * NCH, NCH)], sem))
            for cp in cps:
                cp.wait()
            pltpu.sync_copy(gv, pf_hbm.at[b, c])
            pltpu.sync_copy(ngv, nf_hbm.at[b, c])

            def bump(i, _):
                idxv[pl.ds(i * _L, _L)] = idxv[pl.ds(i * _L, _L)] + HW
                return 0

            lax.fori_loop(0, Q // _L, bump, 0, unroll=True)

            def nbump(i, _):
                nidxv[pl.ds(i * _L, _L)] = nidxv[pl.ds(i * _L, _L)] + HW
                return 0

            lax.fori_loop(0, N // _L, nbump, 0, unroll=True)
            return 0

        lax.fori_loop(0, CPW, c_body, 0)

    return k(f2flat, targ, nidx)


def _tc_qfeat(feat1, B, C, H, W):
    h = _SUB // 2
    nr = H // _SUB
    ncol = W // _SUB
    CB = 8

    selx = (jnp.arange(W)[:, None] == (h + _SUB * jnp.arange(ncol))[None, :]
            ).astype(jnp.float32)
    rr = jnp.arange(CB * nr)
    cc = jnp.arange(CB * H)
    sely = (((cc[None, :] // H) == (rr[:, None] // nr))
            & ((cc[None, :] % H) == (h + _SUB * (rr[:, None] % nr)))
            ).astype(jnp.float32)

    def body(f1_ref, selx_ref, sely_ref, out_ref):
        block = f1_ref[...].reshape(CB * H, W)
        t = lax.dot_general(block, selx_ref[...], (((1,), (0,)), ((), ())),
                            preferred_element_type=jnp.float32)
        qf = lax.dot_general(sely_ref[...], t, (((1,), (0,)), ((), ())),
                             preferred_element_type=jnp.float32)
        out_ref[...] = qf.reshape(1, CB, nr, ncol)

    return pl.pallas_call(
        body,
        grid=(B, C // CB),
        in_specs=[
            pl.BlockSpec((1, CB, H, W), lambda b, c: (b, c, 0, 0)),
            pl.BlockSpec((W, ncol), lambda b, c: (0, 0)),
            pl.BlockSpec((CB * nr, CB * H), lambda b, c: (0, 0)),
        ],
        out_specs=pl.BlockSpec((1, CB, nr, ncol), lambda b, c: (b, c, 0, 0)),
        out_shape=jax.ShapeDtypeStruct((B, C, nr, ncol), jnp.float32),
    )(feat1, selx, sely)


def _tc_loss(qfeat, pfeat, nfeat, qconf, msk):
    B, C, Q = qfeat.shape
    N = nfeat.shape[2]

    def body(q_ref, p_ref, n_ref, qc_ref, m_ref, out_ref, acc_ref):
        bidx = pl.program_id(0)

        @pl.when(bidx == 0)
        def _():
            acc_ref[0] = 0.0
            acc_ref[1] = 0.0

        q = q_ref[0]
        p = p_ref[0]
        n = n_ref[0]
        qn = q / (jnp.sqrt(jnp.sum(q * q, axis=0, keepdims=True)) + 1e-8)
        pn = p / (jnp.sqrt(jnp.sum(p * p, axis=0, keepdims=True)) + 1e-8)
        nn = n / (jnp.sqrt(jnp.sum(n * n, axis=0, keepdims=True)) + 1e-8)
        pos = jnp.sum(qn * pn, axis=0, keepdims=True)
        negT = lax.dot_general(nn, qn, (((0,), (0,)), ((), ())),
                               preferred_element_type=jnp.float32)
        d = jnp.sum(jax.nn.sigmoid((negT - pos) * (1.0 / _ANNEAL)),
                    axis=0, keepdims=True)
        base = 1.0 + _ALPHA * d
        pnp = 1.0 - 1.0 / (base * base)
        qc = qc_ref[0]
        m = m_ref[0]
        pix = pnp * qc + 0.5 * (1.0 - qc)
        acc_ref[0] += jnp.sum(pix * m)
        acc_ref[1] += jnp.sum(m)

        @pl.when(bidx == B - 1)
        def _():
            out_ref[...] = jnp.full((1, 1), acc_ref[0] / (acc_ref[1] + 1e-8),
                                    dtype=jnp.float32)

    out = pl.pallas_call(
        body,
        grid=(B,),
        in_specs=[
            pl.BlockSpec((1, C, Q), lambda b: (b, 0, 0)),
            pl.BlockSpec((1, C, Q), lambda b: (b, 0, 0)),
            pl.BlockSpec((1, C, N), lambda b: (b, 0, 0)),
            pl.BlockSpec((1, 1, Q), lambda b: (b, 0, 0)),
            pl.BlockSpec((1, 1, Q), lambda b: (b, 0, 0)),
        ],
        out_specs=pl.BlockSpec((1, 1), lambda b: (0, 0)),
        out_shape=jax.ShapeDtypeStruct((1, 1), jnp.float32),
        scratch_shapes=[pltpu.SMEM((2,), jnp.float32)],
    )(qfeat, pfeat, nfeat, qconf.reshape(B, 1, Q), msk.reshape(B, 1, Q))
    return out[0, 0]


def kernel(feat1, feat2, reliability, aflow):
    B, C, H, W = feat1.shape
    h = _SUB // 2
    Q = (H // _SUB) * (W // _SUB)

    qconf = reliability[:, 0, h::_SUB, h::_SUB].reshape(B, Q)
    targ = aflow[:, :, h::_SUB, h::_SUB].reshape(B, 2, Q)

    nidx = jax.random.randint(jax.random.key(42), (B, _N_NEG), 0, H * W)
    nidx = nidx.astype(jnp.int32)

    f2flat = _sc_detile(feat2, B, C, H, W).reshape(B * C * H * W)
    pfeat, nfeat, msk = _sc_gather(f2flat, targ, nidx, B, C, H, W, Q)
    qfeat = _tc_qfeat(feat1, B, C, H, W).reshape(B, C, Q)
    return _tc_loss(qfeat, pfeat, nfeat, qconf, msk)


# speedup vs baseline: 10.6146x; 10.6146x over previous
import functools

import jax
import jax.numpy as jnp
from jax import lax
from jax.experimental import pallas as pl
from jax.experimental.pallas import tpu as pltpu
from jax.experimental.pallas import tpu_sc as plsc

_SUB = 8
_N_NEG = 512
_B_PNP = 2.0
_ALPHA = 1.0
_ANNEAL = 0.01
_MAGIC = float(1.5 * (2.0 ** 23))

_L = 16


def _sc_detile(feat2, B, C, H, W):
    mesh = plsc.VectorSubcoreMesh(core_axis_name="c", subcore_axis_name="s")
    NC = mesh.num_cores
    NW = NC * mesh.num_subcores
    P = B * C
    PPW = P // NW

    @functools.partial(
        pl.kernel,
        out_type=[jax.ShapeDtypeStruct((P, H, W), jnp.float32)],
        mesh=mesh,
        scratch_types=[
            pltpu.VMEM((H, W), jnp.float32),
            pltpu.VMEM((H, W), jnp.float32),
            pltpu.SemaphoreType.DMA,
            pltpu.SemaphoreType.DMA,
        ],
    )
    def k(f2_hbm, scr_hbm, bufA, bufB, semI, semO):
        wid = lax.axis_index("s") * NC + lax.axis_index("c")
        p0 = wid * PPW
        bufs = (bufA, bufB)
        cin = [None, None]
        cout = [None, None]
        cin[0] = pltpu.async_copy(
            f2_hbm.at[p0 // C, p0 % C], bufA, semI)
        for pi in range(PPW):
            p = p0 + pi
            par = pi % 2
            if pi + 1 < PPW:
                nxt = p + 1
                if cout[(pi + 1) % 2] is not None:
                    cout[(pi + 1) % 2].wait()
                    cout[(pi + 1) % 2] = None
                cin[(pi + 1) % 2] = pltpu.async_copy(
                    f2_hbm.at[nxt // C, nxt % C], bufs[(pi + 1) % 2], semI)
            cin[par].wait()
            cout[par] = pltpu.async_copy(bufs[par], scr_hbm.at[p], semO)
        for cp in cout:
            if cp is not None:
                cp.wait()

    return k(feat2)[0]


def _sc_gather(f2flat, targ, nidx, B, C, H, W, Q):
    HW = H * W
    N = _N_NEG
    mesh = plsc.VectorSubcoreMesh(core_axis_name="c", subcore_axis_name="s")
    NC = mesh.num_cores
    NW = NC * mesh.num_subcores
    WPB = NW // B
    CPW = C // WPB
    PCH = 112
    NCH = 128

    @functools.partial(
        pl.kernel,
        out_type=[
            jax.ShapeDtypeStruct((B, C, Q), jnp.float32),
            jax.ShapeDtypeStruct((B, C, N), jnp.float32),
            jax.ShapeDtypeStruct((B, Q), jnp.float32),
        ],
        mesh=mesh,
        scratch_types=[
            pltpu.VMEM((Q,), jnp.float32),
            pltpu.VMEM((Q,), jnp.float32),
            pltpu.VMEM((Q,), jnp.float32),
            pltpu.VMEM((Q,), jnp.int32),
            pltpu.VMEM((N,), jnp.int32),
            pltpu.VMEM((Q,), jnp.float32),
            pltpu.VMEM((N,), jnp.float32),
            pltpu.SemaphoreType.DMA,
        ],
    )
    def k(f2_hbm, targ_hbm, nidx_hbm, pf_hbm, nf_hbm, msk_hbm,
          txv, tyv, mskv, idxv, nidxv, gv, ngv, sem):
        wid = lax.axis_index("s") * NC + lax.axis_index("c")
        b = wid // WPB
        cblk = (wid % WPB) * CPW
        base_off = (b * C + cblk) * HW

        pltpu.sync_copy(targ_hbm.at[b, 0], txv)
        pltpu.sync_copy(targ_hbm.at[b, 1], tyv)
        pltpu.sync_copy(nidx_hbm.at[b], nidxv)

        def idx_body(i, _):
            tx = txv[pl.ds(i * _L, _L)]
            ty = tyv[pl.ds(i * _L, _L)]
            rx = (tx + _MAGIC) - _MAGIC
            ry = (ty + _MAGIC) - _MAGIC
            ok = ((rx >= 0.0) & (rx <= W - 1.0)
                  & (ry >= 0.0) & (ry <= H - 1.0))
            cx = jnp.clip(rx, 0.0, W - 1.0).astype(jnp.int32)
            cy = jnp.clip(ry, 0.0, H - 1.0).astype(jnp.int32)
            mskv[pl.ds(i * _L, _L)] = jnp.where(ok, 1.0, 0.0)
            idxv[pl.ds(i * _L, _L)] = cy * W + cx + base_off
            return 0

        lax.fori_loop(0, Q // _L, idx_body, 0, unroll=True)

        def nidx_body(i, _):
            nidxv[pl.ds(i * _L, _L)] = nidxv[pl.ds(i * _L, _L)] + base_off
            return 0

        lax.fori_loop(0, N // _L, nidx_body, 0, unroll=True)

        @pl.when(cblk == 0)
        def _():
            pltpu.sync_copy(mskv, msk_hbm.at[b])

        def c_body(ci, _):
            c = cblk + ci
            cps = []
            for j in range(Q // PCH):
                cps.append(pltpu.async_copy(
                    f2_hbm.at[idxv.at[pl.ds(j * PCH, PCH)]],
                    gv.at[pl.ds(j * PCH, PCH)], sem))
            for j in range(N // NCH):
                cps.append(pltpu.async_copy(
                    f2_hbm.at[nidxv.at[pl.ds(j * NCH, NCH)]],
                    ngv.at[pl.ds(j * NCH, NCH)], sem))
            for cp in cps:
                cp.wait()
            pltpu.sync_copy(gv, pf_hbm.at[b, c])
            pltpu.sync_copy(ngv, nf_hbm.at[b, c])

            def bump(i, _):
                idxv[pl.ds(i * _L, _L)] = idxv[pl.ds(i * _L, _L)] + HW
                return 0

            lax.fori_loop(0, Q // _L, bump, 0, unroll=True)

            def nbump(i, _):
                nidxv[pl.ds(i * _L, _L)] = nidxv[pl.ds(i * _L, _L)] + HW
                return 0

            lax.fori_loop(0, N // _L, nbump, 0, unroll=True)
            return 0

        lax.fori_loop(0, CPW, c_body, 0)

    return k(f2flat, targ, nidx)


def _tc_qfeat(feat1, B, C, H, W):
    h = _SUB // 2
    nr = H // _SUB
    ncol = W // _SUB
    CB = 8

    selx = (jnp.arange(W)[:, None] == (h + _SUB * jnp.arange(ncol))[None, :]
            ).astype(jnp.float32)
    rr = jnp.arange(CB * nr)
    cc = jnp.arange(CB * H)
    sely = (((cc[None, :] // H) == (rr[:, None] // nr))
            & ((cc[None, :] % H) == (h + _SUB * (rr[:, None] % nr)))
            ).astype(jnp.float32)

    def body(f1_ref, selx_ref, sely_ref, out_ref):
        block = f1_ref[...].reshape(CB * H, W)
        t = lax.dot_general(block, selx_ref[...], (((1,), (0,)), ((), ())),
                            preferred_element_type=jnp.float32)
        qf = lax.dot_general(sely_ref[...], t, (((1,), (0,)), ((), ())),
                             preferred_element_type=jnp.float32)
        out_ref[...] = qf.reshape(1, CB, nr, ncol)

    return pl.pallas_call(
        body,
        grid=(B, C // CB),
        in_specs=[
            pl.BlockSpec((1, CB, H, W), lambda b, c: (b, c, 0, 0)),
            pl.BlockSpec((W, ncol), lambda b, c: (0, 0)),
            pl.BlockSpec((CB * nr, CB * H), lambda b, c: (0, 0)),
        ],
        out_specs=pl.BlockSpec((1, CB, nr, ncol), lambda b, c: (b, c, 0, 0)),
        out_shape=jax.ShapeDtypeStruct((B, C, nr, ncol), jnp.float32),
    )(feat1, selx, sely)


def _tc_loss(qfeat, pfeat, nfeat, qconf, msk):
    B, C, Q = qfeat.shape
    N = nfeat.shape[2]

    def body(q_ref, p_ref, n_ref, qc_ref, m_ref, out_ref, acc_ref):
        bidx = pl.program_id(0)

        @pl.when(bidx == 0)
        def _():
            acc_ref[0] = 0.0
            acc_ref[1] = 0.0

        q = q_ref[0]
        p = p_ref[0]
        n = n_ref[0]
        qn = q / (jnp.sqrt(jnp.sum(q * q, axis=0, keepdims=True)) + 1e-8)
        pn = p / (jnp.sqrt(jnp.sum(p * p, axis=0, keepdims=True)) + 1e-8)
        nn = n / (jnp.sqrt(jnp.sum(n * n, axis=0, keepdims=True)) + 1e-8)
        pos = jnp.sum(qn * pn, axis=0, keepdims=True)
        negT = lax.dot_general(nn, qn, (((0,), (0,)), ((), ())),
                               preferred_element_type=jnp.float32)
        d = jnp.sum(jax.nn.sigmoid((negT - pos) * (1.0 / _ANNEAL)),
                    axis=0, keepdims=True)
        base = 1.0 + _ALPHA * d
        pnp = 1.0 - 1.0 / (base * base)
        qc = qc_ref[0]
        m = m_ref[0]
        pix = pnp * qc + 0.5 * (1.0 - qc)
        acc_ref[0] += jnp.sum(pix * m)
        acc_ref[1] += jnp.sum(m)

        @pl.when(bidx == B - 1)
        def _():
            out_ref[...] = jnp.full((1, 1), acc_ref[0] / (acc_ref[1] + 1e-8),
                                    dtype=jnp.float32)

    out = pl.pallas_call(
        body,
        grid=(B,),
        in_specs=[
            pl.BlockSpec((1, C, Q), lambda b: (b, 0, 0)),
            pl.BlockSpec((1, C, Q), lambda b: (b, 0, 0)),
            pl.BlockSpec((1, C, N), lambda b: (b, 0, 0)),
            pl.BlockSpec((1, 1, Q), lambda b: (b, 0, 0)),
            pl.BlockSpec((1, 1, Q), lambda b: (b, 0, 0)),
        ],
        out_specs=pl.BlockSpec((1, 1), lambda b: (0, 0)),
        out_shape=jax.ShapeDtypeStruct((1, 1), jnp.float32),
        scratch_shapes=[pltpu.SMEM((2,), jnp.float32)],
    )(qfeat, pfeat, nfeat, qconf.reshape(B, 1, Q), msk.reshape(B, 1, Q))
    return out[0, 0]


def kernel(feat1, feat2, reliability, aflow):
    B, C, H, W = feat1.shape
    h = _SUB // 2
    Q = (H // _SUB) * (W // _SUB)

    qconf = reliability[:, 0, h::_SUB, h::_SUB].reshape(B, Q)
    targ = aflow[:, :, h::_SUB, h::_SUB].reshape(B, 2, Q)

    nidx = jax.random.randint(jax.random.key(42), (B, _N_NEG), 0, H * W)
    nidx = nidx.astype(jnp.int32)

    f2flat = _sc_detile(feat2, B, C, H, W).reshape(B * C * H * W)
    pfeat, nfeat, msk = _sc_gather(f2flat, targ, nidx, B, C, H, W, Q)
    qfeat = _tc_qfeat(feat1, B, C, H, W).reshape(B, C, Q)
    return _tc_loss(qfeat, pfeat, nfeat, qconf, msk)


# speedup vs baseline: 10.8412x; 1.0214x over previous
import functools

import jax
import jax.numpy as jnp
from jax import lax
from jax.experimental import pallas as pl
from jax.experimental.pallas import tpu as pltpu
from jax.experimental.pallas import tpu_sc as plsc

_SUB = 8
_N_NEG = 512
_B_PNP = 2.0
_ALPHA = 1.0
_ANNEAL = 0.01
_MAGIC = float(1.5 * (2.0 ** 23))

_L = 16


def _sc_detile(feat2, B, C, H, W):
    mesh = plsc.VectorSubcoreMesh(core_axis_name="c", subcore_axis_name="s")
    NC = mesh.num_cores
    NW = NC * mesh.num_subcores
    P = B * C
    PPW = P // NW

    CH = 1
    NCHK = PPW // CH

    @functools.partial(
        pl.kernel,
        out_type=[jax.ShapeDtypeStruct((P, H, W), jnp.float32)],
        mesh=mesh,
        scratch_types=[
            pltpu.VMEM((CH, H, W), jnp.float32),
            pltpu.VMEM((CH, H, W), jnp.float32),
            pltpu.SemaphoreType.DMA,
            pltpu.SemaphoreType.DMA,
        ],
    )
    def k(f2_hbm, scr_hbm, bufA, bufB, semI, semO):
        wid = lax.axis_index("s") * NC + lax.axis_index("c")
        p0 = wid * PPW
        b = p0 // C
        c0 = p0 % C
        bufs = (bufA, bufB)
        cin = [None, None]
        cout = [None, None]
        cin[0] = pltpu.async_copy(f2_hbm.at[b, pl.ds(c0, CH)], bufA, semI)
        for ki in range(NCHK):
            par = ki % 2
            if ki + 1 < NCHK:
                if cout[(ki + 1) % 2] is not None:
                    cout[(ki + 1) % 2].wait()
                    cout[(ki + 1) % 2] = None
                cin[(ki + 1) % 2] = pltpu.async_copy(
                    f2_hbm.at[b, pl.ds(c0 + (ki + 1) * CH, CH)],
                    bufs[(ki + 1) % 2], semI)
            cin[par].wait()
            cout[par] = pltpu.async_copy(
                bufs[par], scr_hbm.at[pl.ds(p0 + ki * CH, CH)], semO)
        for cp in cout:
            if cp is not None:
                cp.wait()

    return k(feat2)[0]


def _sc_gather(f2flat, targ, nidx, B, C, H, W, Q):
    HW = H * W
    N = _N_NEG
    mesh = plsc.VectorSubcoreMesh(core_axis_name="c", subcore_axis_name="s")
    NC = mesh.num_cores
    NW = NC * mesh.num_subcores
    WPB = NW // B
    CPW = C // WPB
    PCH = 112
    NCH = 128

    @functools.partial(
        pl.kernel,
        out_type=[
            jax.ShapeDtypeStruct((B, C, Q), jnp.float32),
            jax.ShapeDtypeStruct((B, C, N), jnp.float32),
            jax.ShapeDtypeStruct((B, Q), jnp.float32),
        ],
        mesh=mesh,
        scratch_types=[
            pltpu.VMEM((Q,), jnp.float32),
            pltpu.VMEM((Q,), jnp.float32),
            pltpu.VMEM((Q,), jnp.float32),
            pltpu.VMEM((Q,), jnp.int32),
            pltpu.VMEM((N,), jnp.int32),
            pltpu.VMEM((Q,), jnp.float32),
            pltpu.VMEM((N,), jnp.float32),
            pltpu.SemaphoreType.DMA,
        ],
    )
    def k(f2_hbm, targ_hbm, nidx_hbm, pf_hbm, nf_hbm, msk_hbm,
          txv, tyv, mskv, idxv, nidxv, gv, ngv, sem):
        wid = lax.axis_index("s") * NC + lax.axis_index("c")
        b = wid // WPB
        cblk = (wid % WPB) * CPW
        base_off = (b * C + cblk) * HW

        pltpu.sync_copy(targ_hbm.at[b, 0], txv)
        pltpu.sync_copy(targ_hbm.at[b, 1], tyv)
        pltpu.sync_copy(nidx_hbm.at[b], nidxv)

        def idx_body(i, _):
            tx = txv[pl.ds(i * _L, _L)]
            ty = tyv[pl.ds(i * _L, _L)]
            rx = (tx + _MAGIC) - _MAGIC
            ry = (ty + _MAGIC) - _MAGIC
            ok = ((rx >= 0.0) & (rx <= W - 1.0)
                  & (ry >= 0.0) & (ry <= H - 1.0))
            cx = jnp.clip(rx, 0.0, W - 1.0).astype(jnp.int32)
            cy = jnp.clip(ry, 0.0, H - 1.0).astype(jnp.int32)
            mskv[pl.ds(i * _L, _L)] = jnp.where(ok, 1.0, 0.0)
            idxv[pl.ds(i * _L, _L)] = cy * W + cx + base_off
            return 0

        lax.fori_loop(0, Q // _L, idx_body, 0, unroll=True)

        def nidx_body(i, _):
            nidxv[pl.ds(i * _L, _L)] = nidxv[pl.ds(i * _L, _L)] + base_off
            return 0

        lax.fori_loop(0, N // _L, nidx_body, 0, unroll=True)

        @pl.when(cblk == 0)
        def _():
            pltpu.sync_copy(mskv, msk_hbm.at[b])

        def c_body(ci, _):
            c = cblk + ci
            cps = []
            for j in range(Q // PCH):
                cps.append(pltpu.async_copy(
                    f2_hbm.at[idxv.at[pl.ds(j * PCH, PCH)]],
                    gv.at[pl.ds(j * PCH, PCH)], sem))
            for j in range(N // NCH):
                cps.append(pltpu.async_copy(
                    f2_hbm.at[nidxv.at[pl.ds(j * NCH, NCH)]],
                    ngv.at[pl.ds(j * NCH, NCH)], sem))
            for cp in cps:
                cp.wait()
            pltpu.sync_copy(gv, pf_hbm.at[b, c])
            pltpu.sync_copy(ngv, nf_hbm.at[b, c])

            def bump(i, _):
                idxv[pl.ds(i * _L, _L)] = idxv[pl.ds(i * _L, _L)] + HW
                return 0

            lax.fori_loop(0, Q // _L, bump, 0, unroll=True)

            def nbump(i, _):
                nidxv[pl.ds(i * _L, _L)] = nidxv[pl.ds(i * _L, _L)] + HW
                return 0

            lax.fori_loop(0, N // _L, nbump, 0, unroll=True)
            return 0

        lax.fori_loop(0, CPW, c_body, 0)

    return k(f2flat, targ, nidx)


def _tc_qfeat(feat1, B, C, H, W):
    h = _SUB // 2
    nr = H // _SUB
    ncol = W // _SUB
    CB = 16

    selx = (jnp.arange(W)[:, None] == (h + _SUB * jnp.arange(ncol))[None, :]
            ).astype(jnp.float32)
    rr = jnp.arange(CB * nr)
    cc = jnp.arange(CB * H)
    sely = (((cc[None, :] // H) == (rr[:, None] // nr))
            & ((cc[None, :] % H) == (h + _SUB * (rr[:, None] % nr)))
            ).astype(jnp.float32)

    def body(f1_ref, selx_ref, sely_ref, out_ref):
        block = f1_ref[...].reshape(CB * H, W)
        t = lax.dot_general(block, selx_ref[...], (((1,), (0,)), ((), ())),
                            preferred_element_type=jnp.float32)
        qf = lax.dot_general(sely_ref[...], t, (((1,), (0,)), ((), ())),
                             preferred_element_type=jnp.float32)
        out_ref[...] = qf.reshape(1, CB, nr, ncol)

    return pl.pallas_call(
        body,
        grid=(B, C // CB),
        in_specs=[
            pl.BlockSpec((1, CB, H, W), lambda b, c: (b, c, 0, 0)),
            pl.BlockSpec((W, ncol), lambda b, c: (0, 0)),
            pl.BlockSpec((CB * nr, CB * H), lambda b, c: (0, 0)),
        ],
        out_specs=pl.BlockSpec((1, CB, nr, ncol), lambda b, c: (b, c, 0, 0)),
        out_shape=jax.ShapeDtypeStruct((B, C, nr, ncol), jnp.float32),
    )(feat1, selx, sely)


def _tc_loss(qfeat, pfeat, nfeat, qconf, msk):
    B, C, Q = qfeat.shape
    N = nfeat.shape[2]

    def body(q_ref, p_ref, n_ref, qc_ref, m_ref, out_ref, acc_ref):
        bidx = pl.program_id(0)

        @pl.when(bidx == 0)
        def _():
            acc_ref[0] = 0.0
            acc_ref[1] = 0.0

        q = q_ref[0]
        p = p_ref[0]
        n = n_ref[0]
        qn = q / (jnp.sqrt(jnp.sum(q * q, axis=0, keepdims=True)) + 1e-8)
        pn = p / (jnp.sqrt(jnp.sum(p * p, axis=0, keepdims=True)) + 1e-8)
        nn = n / (jnp.sqrt(jnp.sum(n * n, axis=0, keepdims=True)) + 1e-8)
        pos = jnp.sum(qn * pn, axis=0, keepdims=True)
        negT = lax.dot_general(nn, qn, (((0,), (0,)), ((), ())),
                               preferred_element_type=jnp.float32)
        d = jnp.sum(jax.nn.sigmoid((negT - pos) * (1.0 / _ANNEAL)),
                    axis=0, keepdims=True)
        base = 1.0 + _ALPHA * d
        pnp = 1.0 - 1.0 / (base * base)
        qc = qc_ref[0]
        m = m_ref[0]
        pix = pnp * qc + 0.5 * (1.0 - qc)
        acc_ref[0] += jnp.sum(pix * m)
        acc_ref[1] += jnp.sum(m)

        @pl.when(bidx == B - 1)
        def _():
            out_ref[...] = jnp.full((1, 1), acc_ref[0] / (acc_ref[1] + 1e-8),
                                    dtype=jnp.float32)

    out = pl.pallas_call(
        body,
        grid=(B,),
        in_specs=[
            pl.BlockSpec((1, C, Q), lambda b: (b, 0, 0)),
            pl.BlockSpec((1, C, Q), lambda b: (b, 0, 0)),
            pl.BlockSpec((1, C, N), lambda b: (b, 0, 0)),
            pl.BlockSpec((1, 1, Q), lambda b: (b, 0, 0)),
            pl.BlockSpec((1, 1, Q), lambda b: (b, 0, 0)),
        ],
        out_specs=pl.BlockSpec((1, 1), lambda b: (0, 0)),
        out_shape=jax.ShapeDtypeStruct((1, 1), jnp.float32),
        scratch_shapes=[pltpu.SMEM((2,), jnp.float32)],
    )(qfeat, pfeat, nfeat, qconf.reshape(B, 1, Q), msk.reshape(B, 1, Q))
    return out[0, 0]


def kernel(feat1, feat2, reliability, aflow):
    B, C, H, W = feat1.shape
    h = _SUB // 2
    Q = (H // _SUB) * (W // _SUB)

    qconf = reliability[:, 0, h::_SUB, h::_SUB].reshape(B, Q)
    targ = aflow[:, :, h::_SUB, h::_SUB].reshape(B, 2, Q)

    nidx = jax.random.randint(jax.random.key(42), (B, _N_NEG), 0, H * W)
    nidx = nidx.astype(jnp.int32)

    f2flat = _sc_detile(feat2, B, C, H, W).reshape(B * C * H * W)
    pfeat, nfeat, msk = _sc_gather(f2flat, targ, nidx, B, C, H, W, Q)
    qfeat = _tc_qfeat(feat1, B, C, H, W).reshape(B, C, Q)
    return _tc_loss(qfeat, pfeat, nfeat, qconf, msk)


# speedup vs baseline: 14.9667x; 1.3805x over previous
import functools

import jax
import jax.numpy as jnp
from jax import lax
from jax.experimental import pallas as pl
from jax.experimental.pallas import tpu as pltpu
from jax.experimental.pallas import tpu_sc as plsc

_SUB = 8
_N_NEG = 512
_B_PNP = 2.0
_ALPHA = 1.0
_ANNEAL = 0.01
_MAGIC = float(1.5 * (2.0 ** 23))

_L = 16


def _sc_gather(f2flat, targ, nidx, B, C, H, W, Q):
    HW = H * W
    N = _N_NEG
    mesh = plsc.VectorSubcoreMesh(core_axis_name="c", subcore_axis_name="s")
    NC = mesh.num_cores
    NW = NC * mesh.num_subcores
    WPB = NW // B
    CPW = C // WPB
    PCH = 112
    NCH = 128

    @functools.partial(
        pl.kernel,
        out_type=[
            jax.ShapeDtypeStruct((B, C, Q), jnp.float32),
            jax.ShapeDtypeStruct((B, C, N), jnp.float32),
            jax.ShapeDtypeStruct((B, Q), jnp.float32),
        ],
        mesh=mesh,
        scratch_types=[
            pltpu.VMEM((Q,), jnp.float32),
            pltpu.VMEM((Q,), jnp.float32),
            pltpu.VMEM((Q,), jnp.float32),
            pltpu.VMEM((Q,), jnp.int32),
            pltpu.VMEM((N,), jnp.int32),
            pltpu.VMEM((Q,), jnp.float32),
            pltpu.VMEM((N,), jnp.float32),
            pltpu.SemaphoreType.DMA,
        ],
    )
    def k(f2_hbm, targ_hbm, nidx_hbm, pf_hbm, nf_hbm, msk_hbm,
          txv, tyv, mskv, idxv, nidxv, gv, ngv, sem):
        wid = lax.axis_index("s") * NC + lax.axis_index("c")
        b = wid // WPB
        cblk = (wid % WPB) * CPW
        base_off = (b * C + cblk) * HW

        pltpu.sync_copy(targ_hbm.at[b, 0], txv)
        pltpu.sync_copy(targ_hbm.at[b, 1], tyv)
        pltpu.sync_copy(nidx_hbm.at[b], nidxv)

        def idx_body(i, _):
            tx = txv[pl.ds(i * _L, _L)]
            ty = tyv[pl.ds(i * _L, _L)]
            rx = (tx + _MAGIC) - _MAGIC
            ry = (ty + _MAGIC) - _MAGIC
            ok = ((rx >= 0.0) & (rx <= W - 1.0)
                  & (ry >= 0.0) & (ry <= H - 1.0))
            cx = jnp.clip(rx, 0.0, W - 1.0).astype(jnp.int32)
            cy = jnp.clip(ry, 0.0, H - 1.0).astype(jnp.int32)
            mskv[pl.ds(i * _L, _L)] = jnp.where(ok, 1.0, 0.0)
            idxv[pl.ds(i * _L, _L)] = cy * W + cx + base_off
            return 0

        lax.fori_loop(0, Q // _L, idx_body, 0, unroll=True)

        def nidx_body(i, _):
            nidxv[pl.ds(i * _L, _L)] = nidxv[pl.ds(i * _L, _L)] + base_off
            return 0

        lax.fori_loop(0, N // _L, nidx_body, 0, unroll=True)

        @pl.when(cblk == 0)
        def _():
            pltpu.sync_copy(mskv, msk_hbm.at[b])

        def c_body(ci, _):
            c = cblk + ci
            cps = []
            for j in range(Q // PCH):
                cps.append(pltpu.async_copy(
                    f2_hbm.at[idxv.at[pl.ds(j * PCH, PCH)]],
                    gv.at[pl.ds(j * PCH, PCH)], sem))
            for j in range(N // NCH):
                cps.append(pltpu.async_copy(
                    f2_hbm.at[nidxv.at[pl.ds(j * NCH, NCH)]],
                    ngv.at[pl.ds(j * NCH, NCH)], sem))
            for cp in cps:
                cp.wait()
            pltpu.sync_copy(gv, pf_hbm.at[b, c])
            pltpu.sync_copy(ngv, nf_hbm.at[b, c])

            def bump(i, _):
                idxv[pl.ds(i * _L, _L)] = idxv[pl.ds(i * _L, _L)] + HW
                return 0

            lax.fori_loop(0, Q // _L, bump, 0, unroll=True)

            def nbump(i, _):
                nidxv[pl.ds(i * _L, _L)] = nidxv[pl.ds(i * _L, _L)] + HW
                return 0

            lax.fori_loop(0, N // _L, nbump, 0, unroll=True)
            return 0

        lax.fori_loop(0, CPW, c_body, 0)

    return k(f2flat, targ, nidx)


def _tc_qfeat(feat1, B, C, H, W):
    h = _SUB // 2
    nr = H // _SUB
    ncol = W // _SUB
    CB = 16

    selx = (jnp.arange(W)[:, None] == (h + _SUB * jnp.arange(ncol))[None, :]
            ).astype(jnp.float32)

    def body(f1_ref, selx_ref, out_ref):
        rows = f1_ref[...].reshape(CB * nr, _SUB, W)[:, h, :]
        qf = lax.dot_general(rows, selx_ref[...], (((1,), (0,)), ((), ())),
                             preferred_element_type=jnp.float32)
        out_ref[...] = qf.reshape(1, CB, nr, ncol)

    return pl.pallas_call(
        body,
        grid=(B, C // CB),
        in_specs=[
            pl.BlockSpec((1, CB, H, W), lambda b, c: (b, c, 0, 0)),
            pl.BlockSpec((W, ncol), lambda b, c: (0, 0)),
        ],
        out_specs=pl.BlockSpec((1, CB, nr, ncol), lambda b, c: (b, c, 0, 0)),
        out_shape=jax.ShapeDtypeStruct((B, C, nr, ncol), jnp.float32),
    )(feat1, selx)


def _tc_loss(qfeat, pfeat, nfeat, qconf, msk):
    B, C, Q = qfeat.shape
    N = nfeat.shape[2]

    def body(q_ref, p_ref, n_ref, qc_ref, m_ref, out_ref, acc_ref):
        bidx = pl.program_id(0)

        @pl.when(bidx == 0)
        def _():
            acc_ref[0] = 0.0
            acc_ref[1] = 0.0

        q = q_ref[0]
        p = p_ref[0]
        n = n_ref[0]
        qn = q / (jnp.sqrt(jnp.sum(q * q, axis=0, keepdims=True)) + 1e-8)
        pn = p / (jnp.sqrt(jnp.sum(p * p, axis=0, keepdims=True)) + 1e-8)
        nn = n / (jnp.sqrt(jnp.sum(n * n, axis=0, keepdims=True)) + 1e-8)
        pos = jnp.sum(qn * pn, axis=0, keepdims=True)
        negT = lax.dot_general(nn, qn, (((0,), (0,)), ((), ())),
                               preferred_element_type=jnp.float32)
        d = jnp.sum(jax.nn.sigmoid((negT - pos) * (1.0 / _ANNEAL)),
                    axis=0, keepdims=True)
        base = 1.0 + _ALPHA * d
        pnp = 1.0 - 1.0 / (base * base)
        qc = qc_ref[0]
        m = m_ref[0]
        pix = pnp * qc + 0.5 * (1.0 - qc)
        acc_ref[0] += jnp.sum(pix * m)
        acc_ref[1] += jnp.sum(m)

        @pl.when(bidx == B - 1)
        def _():
            out_ref[...] = jnp.full((1, 1), acc_ref[0] / (acc_ref[1] + 1e-8),
                                    dtype=jnp.float32)

    out = pl.pallas_call(
        body,
        grid=(B,),
        in_specs=[
            pl.BlockSpec((1, C, Q), lambda b: (b, 0, 0)),
            pl.BlockSpec((1, C, Q), lambda b: (b, 0, 0)),
            pl.BlockSpec((1, C, N), lambda b: (b, 0, 0)),
            pl.BlockSpec((1, 1, Q), lambda b: (b, 0, 0)),
            pl.BlockSpec((1, 1, Q), lambda b: (b, 0, 0)),
        ],
        out_specs=pl.BlockSpec((1, 1), lambda b: (0, 0)),
        out_shape=jax.ShapeDtypeStruct((1, 1), jnp.float32),
        scratch_shapes=[pltpu.SMEM((2,), jnp.float32)],
    )(qfeat, pfeat, nfeat, qconf.reshape(B, 1, Q), msk.reshape(B, 1, Q))
    return out[0, 0]


def kernel(feat1, feat2, reliability, aflow):
    B, C, H, W = feat1.shape
    h = _SUB // 2
    Q = (H // _SUB) * (W // _SUB)

    qconf = reliability[:, 0, h::_SUB, h::_SUB].reshape(B, Q)
    targ = aflow[:, :, h::_SUB, h::_SUB].reshape(B, 2, Q)

    nidx = jax.random.randint(jax.random.key(42), (B, _N_NEG), 0, H * W)
    nidx = nidx.astype(jnp.int32)

    f2flat = feat2.reshape(B * C * H * W)
    pfeat, nfeat, msk = _sc_gather(f2flat, targ, nidx, B, C, H, W, Q)
    qfeat = _tc_qfeat(feat1, B, C, H, W).reshape(B, C, Q)
    return _tc_loss(qfeat, pfeat, nfeat, qconf, msk)
